# Initial kernel scaffold; baseline (speedup 1.0000x reference)
import jax
import jax.numpy as jnp
from jax.experimental import pallas as pl


def kernel(rois, probs, deltas, masks, window):
    raise NotImplementedError("write your pallas kernel here")


# speedup vs baseline: 62.4200x; 62.4200x over previous
import functools

import jax
import jax.numpy as jnp
from jax import lax
from jax.experimental import pallas as pl
from jax.experimental.pallas import tpu as pltpu
from jax.experimental.pallas import tpu_sc as plsc

B = 2
N = 1000
NP = 1024
C = 81
MH = 14
MW = 14
MROW = MH * MW * C
MAX_OUT = 100
NMS_THR = 0.3
MIN_CONF = 0.5
STD = (0.1, 0.1, 0.2, 0.2)


def _tc_body(win_ref, probs_ref, deltas_ref, rois_ref, det_ref, top_ref):
    wy1 = win_ref[0]
    wx1 = win_ref[1]
    wy2 = win_ref[2]
    wx2 = win_ref[3]

    best = probs_ref[0, 0]
    cid = jnp.zeros((8, 128), jnp.int32)
    d0 = deltas_ref[0, 0 * C + 0]
    d1 = deltas_ref[0, 1 * C + 0]
    d2 = deltas_ref[0, 2 * C + 0]
    d3 = deltas_ref[0, 3 * C + 0]
    for c in range(1, C):
        pc = probs_ref[0, c]
        upd = pc > best
        best = jnp.where(upd, pc, best)
        cid = jnp.where(upd, c, cid)
        d0 = jnp.where(upd, deltas_ref[0, 0 * C + c], d0)
        d1 = jnp.where(upd, deltas_ref[0, 1 * C + c], d1)
        d2 = jnp.where(upd, deltas_ref[0, 2 * C + c], d2)
        d3 = jnp.where(upd, deltas_ref[0, 3 * C + c], d3)

    ry1 = rois_ref[0, 0]
    rx1 = rois_ref[0, 1]
    ry2 = rois_ref[0, 2]
    rx2 = rois_ref[0, 3]
    nz = (jnp.abs(ry1) + jnp.abs(rx1) + jnp.abs(ry2) + jnp.abs(rx2)) != 0.0

    height = ry2 - ry1
    width = rx2 - rx1
    cy = ry1 + 0.5 * height + (d0 * STD[0]) * height
    cx = rx1 + 0.5 * width + (d1 * STD[1]) * width
    height = height * jnp.exp(d2 * STD[2])
    width = width * jnp.exp(d3 * STD[3])
    y1 = cy - 0.5 * height
    x1 = cx - 0.5 * width
    y2 = y1 + height
    x2 = x1 + width
    y1 = jnp.clip(y1, wy1, wy2)
    x1 = jnp.clip(x1, wx1, wx2)
    y2 = jnp.clip(y2, wy1, wy2)
    x2 = jnp.clip(x2, wx1, wx2)
    area = (y2 - y1) * (x2 - x1)

    valid = nz & (cid > 0) & (best >= MIN_CONF)
    key0 = jnp.where(valid, best, -1.0)

    lin = (
        lax.broadcasted_iota(jnp.int32, (8, 128), 0) * 128
        + lax.broadcasted_iota(jnp.int32, (8, 128), 1)
    )
    lane = lax.broadcasted_iota(jnp.int32, (1, 128), 1)

    def step(t, carry):
        key, top, cnt = carry
        m = jnp.max(key)
        alive = m >= 0.0
        idx = jnp.min(jnp.where(key == m, lin, NP))
        sel = lin == idx
        gy1 = jnp.sum(jnp.where(sel, y1, 0.0))
        gx1 = jnp.sum(jnp.where(sel, x1, 0.0))
        gy2 = jnp.sum(jnp.where(sel, y2, 0.0))
        gx2 = jnp.sum(jnp.where(sel, x2, 0.0))
        ga = jnp.sum(jnp.where(sel, area, 0.0))
        gs = jnp.sum(jnp.where(sel, best, 0.0))
        gc = jnp.sum(jnp.where(sel, cid, 0))
        yy1 = jnp.maximum(gy1, y1)
        xx1 = jnp.maximum(gx1, x1)
        yy2 = jnp.minimum(gy2, y2)
        xx2 = jnp.minimum(gx2, x2)
        inter = jnp.maximum(0.0, yy2 - yy1) * jnp.maximum(0.0, xx2 - xx1)
        iou = inter / (ga + area - inter + 1e-12)
        supp = (cid == gc) & (iou > NMS_THR)
        key = jnp.where(alive, jnp.where(supp | sel, -1.0, key), key)
        top = jnp.where(alive & (lin == t), idx, top)
        cnt = cnt + jnp.where(alive, 1, 0)
        row = jnp.where(
            lane == 0, gy1,
            jnp.where(lane == 1, gx1,
                      jnp.where(lane == 2, gy2,
                                jnp.where(lane == 3, gx2,
                                          jnp.where(lane == 4, gc.astype(jnp.float32),
                                                    jnp.where(lane == 5, gs, 0.0))))))
        row = jnp.where(alive, row, 0.0)
        det_ref[0, pl.ds(t, 1), :] = row
        return key, top, cnt

    top0 = jnp.zeros((8, 128), jnp.int32)
    _, top, cnt = lax.fori_loop(0, MAX_OUT, step, (key0, top0, 0))
    top = jnp.where(lin == NP - 1, cnt, top)
    top_ref[0] = top


def _tc_call(probs_t, deltas_t, rois_t, window):
    return pl.pallas_call(
        _tc_body,
        grid=(B,),
        in_specs=[
            pl.BlockSpec(memory_space=pltpu.SMEM),
            pl.BlockSpec((1, C, 8, 128), lambda b: (b, 0, 0, 0)),
            pl.BlockSpec((1, 4 * C, 8, 128), lambda b: (b, 0, 0, 0)),
            pl.BlockSpec((1, 4, 8, 128), lambda b: (b, 0, 0, 0)),
        ],
        out_specs=[
            pl.BlockSpec((1, MAX_OUT, 128), lambda b: (b, 0, 0)),
            pl.BlockSpec((1, 8, 128), lambda b: (b, 0, 0)),
        ],
        out_shape=[
            jax.ShapeDtypeStruct((B, MAX_OUT, 128), jnp.float32),
            jax.ShapeDtypeStruct((B, 8, 128), jnp.int32),
        ],
    )(window, probs_t, deltas_t, rois_t)


_NTILES = 25
_RPT = 8


def _sc_body(masks_hbm, idx_hbm, scale_hbm, zeros_hbm, out_hbm, idx_v, scale_v, row_v):
    wid = lax.axis_index("s") * 2 + lax.axis_index("c")

    @pl.when(wid < _NTILES)
    def _():
        base = wid * _RPT
        pltpu.sync_copy(idx_hbm.at[pl.ds(base, _RPT)], idx_v.at[pl.ds(0, _RPT)])
        pltpu.sync_copy(scale_hbm.at[pl.ds(base, _RPT)], scale_v)
        ivec = idx_v[...]
        iota = lax.iota(jnp.int32, 16)
        for j in range(_RPT):
            s = jnp.max(scale_v[j])
            n = jnp.max(jnp.where(iota == j, ivec, 0))

            @pl.when(s > 0.5)
            def _():
                pltpu.sync_copy(masks_hbm.at[pl.ds(n * MH, MH)], row_v)

            @pl.when(s < 0.5)
            def _():
                pltpu.sync_copy(zeros_hbm, row_v)

            pltpu.sync_copy(row_v, out_hbm.at[base + j])


@functools.lru_cache(maxsize=1)
def _sc_gather_fn():
    return pl.kernel(
        _sc_body,
        out_type=jax.ShapeDtypeStruct((B * MAX_OUT, MH, MW, C), jnp.float32),
        mesh=plsc.VectorSubcoreMesh(core_axis_name="c", subcore_axis_name="s"),
        compiler_params=pltpu.CompilerParams(needs_layout_passes=False),
        scratch_types=[
            pltpu.VMEM((16,), jnp.int32),
            pltpu.VMEM((_RPT, 16), jnp.float32),
            pltpu.VMEM((MH, MW, C), jnp.float32),
        ],
    )


@jax.jit
def kernel(rois, probs, deltas, masks, window):
    probs_p = jnp.pad(probs, ((0, 0), (0, NP - N), (0, 0)))
    probs_t = probs_p.transpose(0, 2, 1).reshape(B, C, 8, 128)
    deltas_p = jnp.pad(deltas, ((0, 0), (0, NP - N), (0, 0), (0, 0)))
    deltas_t = deltas_p.transpose(0, 3, 2, 1).reshape(B, 4 * C, 8, 128)
    rois_t = (
        jnp.pad(rois, ((0, 0), (0, NP - N), (0, 0)))
        .transpose(0, 2, 1)
        .reshape(B, 4, 8, 128)
    )

    det_out, top_out = _tc_call(probs_t, deltas_t, rois_t, window)

    dets = det_out[:, :, :6]
    top_flat = top_out.reshape(B, NP)
    nvalid = top_flat[:, NP - 1]
    tsel = top_flat[:, :MAX_OUT]
    gidx = (tsel + jnp.arange(B, dtype=jnp.int32)[:, None] * N).reshape(B * MAX_OUT)
    scale = (
        jnp.arange(MAX_OUT, dtype=jnp.int32)[None, :] < nvalid[:, None]
    ).astype(jnp.float32).reshape(B * MAX_OUT)
    scale_b = jnp.broadcast_to(scale[:, None], (B * MAX_OUT, 16))

    masks_view = masks.reshape(B * N * MH, MW, C)
    zeros = jnp.zeros((MH, MW, C), jnp.float32)
    out = _sc_gather_fn()(masks_view, gidx, scale_b, zeros)
    return dets, out.reshape(B, MAX_OUT, MH, MW, C)


# speedup vs baseline: 192.6014x; 3.0856x over previous
import functools

import jax
import jax.numpy as jnp
from jax import lax
from jax.experimental import pallas as pl
from jax.experimental.pallas import tpu as pltpu
from jax.experimental.pallas import tpu_sc as plsc

B = 2
N = 1000
C = 81
MH = 14
MW = 14
MROW = MH * MW * C
MAX_OUT = 100
NMS_THR = 0.3
MIN_CONF = 0.5
STD = (0.1, 0.1, 0.2, 0.2)


def _tc_body(win_ref, probs_ref, deltas_ref, rois_ref, det_ref, top_ref):
    wy1 = win_ref[0]
    wx1 = win_ref[1]
    wy2 = win_ref[2]
    wx2 = win_ref[3]

    best = probs_ref[0]
    cid = jnp.zeros((B, N), jnp.int32)
    d0 = deltas_ref[:, 0, 0, :]
    d1 = deltas_ref[:, 0, 1, :]
    d2 = deltas_ref[:, 0, 2, :]
    d3 = deltas_ref[:, 0, 3, :]
    for c in range(1, C):
        pc = probs_ref[c]
        upd = pc > best
        best = jnp.where(upd, pc, best)
        cid = jnp.where(upd, c, cid)
        d0 = jnp.where(upd, deltas_ref[:, c, 0, :], d0)
        d1 = jnp.where(upd, deltas_ref[:, c, 1, :], d1)
        d2 = jnp.where(upd, deltas_ref[:, c, 2, :], d2)
        d3 = jnp.where(upd, deltas_ref[:, c, 3, :], d3)

    ry1 = rois_ref[:, 0, :]
    rx1 = rois_ref[:, 1, :]
    ry2 = rois_ref[:, 2, :]
    rx2 = rois_ref[:, 3, :]
    nz = (jnp.abs(ry1) + jnp.abs(rx1) + jnp.abs(ry2) + jnp.abs(rx2)) != 0.0

    height = ry2 - ry1
    width = rx2 - rx1
    cy = ry1 + 0.5 * height + (d0 * STD[0]) * height
    cx = rx1 + 0.5 * width + (d1 * STD[1]) * width
    height = height * jnp.exp(d2 * STD[2])
    width = width * jnp.exp(d3 * STD[3])
    y1 = cy - 0.5 * height
    x1 = cx - 0.5 * width
    y2 = y1 + height
    x2 = x1 + width
    y1 = jnp.clip(y1, wy1, wy2)
    x1 = jnp.clip(x1, wx1, wx2)
    y2 = jnp.clip(y2, wy1, wy2)
    x2 = jnp.clip(x2, wx1, wx2)
    area = (y2 - y1) * (x2 - x1)

    valid = nz & (cid > 0) & (best >= MIN_CONF)
    key0 = jnp.where(valid, best, -1.0)

    lin = lax.broadcasted_iota(jnp.int32, (B, N), 1)
    lane = lax.broadcasted_iota(jnp.int32, (B, 1, 128), 2)

    def step(t, carry):
        key, top, cnt = carry
        m = jnp.max(key, axis=1, keepdims=True)
        alive = m >= 0.0
        idx = jnp.min(jnp.where(key == m, lin, N), axis=1, keepdims=True)
        sel = lin == idx
        gy1 = jnp.sum(jnp.where(sel, y1, 0.0), axis=1, keepdims=True)
        gx1 = jnp.sum(jnp.where(sel, x1, 0.0), axis=1, keepdims=True)
        gy2 = jnp.sum(jnp.where(sel, y2, 0.0), axis=1, keepdims=True)
        gx2 = jnp.sum(jnp.where(sel, x2, 0.0), axis=1, keepdims=True)
        ga = jnp.sum(jnp.where(sel, area, 0.0), axis=1, keepdims=True)
        gs = jnp.sum(jnp.where(sel, best, 0.0), axis=1, keepdims=True)
        gc = jnp.sum(jnp.where(sel, cid, 0), axis=1, keepdims=True)
        yy1 = jnp.maximum(gy1, y1)
        xx1 = jnp.maximum(gx1, x1)
        yy2 = jnp.minimum(gy2, y2)
        xx2 = jnp.minimum(gx2, x2)
        inter = jnp.maximum(0.0, yy2 - yy1) * jnp.maximum(0.0, xx2 - xx1)
        iou = inter / (ga + area - inter + 1e-12)
        supp = (cid == gc) & (iou > NMS_THR)
        key = jnp.where(alive, jnp.where(supp | sel, -1.0, key), key)
        top = jnp.where(alive & (lin == t), idx, top)
        cnt = cnt + jnp.where(alive, 1, 0)
        row = jnp.where(
            lane == 0, gy1[:, :, None],
            jnp.where(lane == 1, gx1[:, :, None],
                      jnp.where(lane == 2, gy2[:, :, None],
                                jnp.where(lane == 3, gx2[:, :, None],
                                          jnp.where(lane == 4,
                                                    gc.astype(jnp.float32)[:, :, None],
                                                    jnp.where(lane == 5,
                                                              gs[:, :, None], 0.0))))))
        row = jnp.where(alive[:, :, None], row, 0.0)
        det_ref[:, pl.ds(t, 1), :] = row
        return key, top, cnt

    top0 = jnp.zeros((B, N), jnp.int32)
    cnt0 = jnp.zeros((B, 1), jnp.int32)
    _, top, cnt = lax.fori_loop(0, MAX_OUT, step, (key0, top0, cnt0))
    det_ref[:, pl.ds(MAX_OUT, 1), :] = (
        cnt.astype(jnp.float32)[:, :, None] + jnp.zeros((B, 1, 128), jnp.float32)
    )
    top_ref[:, pl.ds(0, 1), :] = top[:, None, :]


def _tc_call(probs_t, deltas_t, rois_t, window):
    return pl.pallas_call(
        _tc_body,
        in_specs=[
            pl.BlockSpec(memory_space=pltpu.SMEM),
            pl.BlockSpec((C, B, N), lambda: (0, 0, 0)),
            pl.BlockSpec((B, C, 4, N), lambda: (0, 0, 0, 0)),
            pl.BlockSpec((B, 4, N), lambda: (0, 0, 0)),
        ],
        out_specs=[
            pl.BlockSpec((B, MAX_OUT + 1, 128), lambda: (0, 0, 0)),
            pl.BlockSpec((B, 1, N), lambda: (0, 0, 0)),
        ],
        out_shape=[
            jax.ShapeDtypeStruct((B, MAX_OUT + 1, 128), jnp.float32),
            jax.ShapeDtypeStruct((B, 1, N), jnp.int32),
        ],
    )(window, probs_t, deltas_t, rois_t)


_OFFS = (0, 16, 32, 48, 64, 80, 84)
_NCH = len(_OFFS)
_BLK = 4
_NBLK = MROW // _BLK
_BPT = _NBLK // 32


def _sc_body(src, idx_hbm, scl_hbm, out, idxv, sclv, ib0, ib1, ob0, ob1,
             tin, tob, si0, si1, so0, so1):
    wid = lax.axis_index("s") * 2 + lax.axis_index("c")
    pltpu.sync_copy(idx_hbm, idxv)
    pltpu.sync_copy(scl_hbm, sclv)
    iota = lax.iota(jnp.int32, 16)
    zero = jnp.zeros((16,), jnp.int32)

    def process(ib, ob):
        for r in range(_BLK):
            rv = zero + r
            for b in range(B):
                bv = zero + b
                for j in range(_NCH):
                    ii = idxv[b * _NCH + j]
                    sc = sclv[b * _NCH + j]
                    g = plsc.load_gather(ib, [rv, bv, ii])
                    plsc.store_scatter(ob, [rv, bv, _OFFS[j] + iota], g * sc)

    base0 = wid * _BPT * _BLK
    pltpu.make_async_copy(src.at[pl.ds(base0, _BLK)], ib0, si0).start()
    pltpu.make_async_copy(src.at[pl.ds(base0 + _BLK, _BLK)], ib1, si1).start()

    def outer(i, _):
        for ph, ib, ob, si, so in ((0, ib0, ob0, si0, so0), (1, ib1, ob1, si1, so1)):
            j = 2 * i + ph
            rbase = base0 + j * _BLK
            pltpu.make_async_copy(src.at[pl.ds(rbase, _BLK)], ib, si).wait()

            @pl.when(j >= 2)
            def _():
                pltpu.make_async_copy(
                    ob, out.at[pl.ds(rbase - 2 * _BLK, _BLK)], so).wait()

            process(ib, ob)
            pltpu.make_async_copy(ob, out.at[pl.ds(rbase, _BLK)], so).start()

            @pl.when(j + 2 < _BPT)
            def _():
                pltpu.make_async_copy(
                    src.at[pl.ds(rbase + 2 * _BLK, _BLK)], ib, si).start()

        return 0

    lax.fori_loop(0, _BPT // 2, outer, 0)
    last = base0 + (_BPT - 2) * _BLK
    pltpu.make_async_copy(ob0, out.at[pl.ds(last, _BLK)], so0).wait()
    pltpu.make_async_copy(ob1, out.at[pl.ds(last + _BLK, _BLK)], so1).wait()

    @pl.when(wid == 0)
    def _():
        tbase = 32 * _BPT * _BLK
        pltpu.sync_copy(src.at[pl.ds(tbase, _BLK)], tin)
        process(tin, tob)
        pltpu.sync_copy(tob, out.at[pl.ds(tbase, _BLK)])


@functools.lru_cache(maxsize=1)
def _sc_gather_fn():
    return pl.kernel(
        _sc_body,
        out_type=jax.ShapeDtypeStruct((MROW, B, MAX_OUT), jnp.float32),
        mesh=plsc.VectorSubcoreMesh(core_axis_name="c", subcore_axis_name="s"),
        compiler_params=pltpu.CompilerParams(needs_layout_passes=False),
        scratch_types=[
            pltpu.VMEM((B * _NCH, 16), jnp.int32),
            pltpu.VMEM((B * _NCH, 16), jnp.float32),
            pltpu.VMEM((_BLK, B, N), jnp.float32),
            pltpu.VMEM((_BLK, B, N), jnp.float32),
            pltpu.VMEM((_BLK, B, MAX_OUT), jnp.float32),
            pltpu.VMEM((_BLK, B, MAX_OUT), jnp.float32),
            pltpu.VMEM((_BLK, B, N), jnp.float32),
            pltpu.VMEM((_BLK, B, MAX_OUT), jnp.float32),
            pltpu.SemaphoreType.DMA,
            pltpu.SemaphoreType.DMA,
            pltpu.SemaphoreType.DMA,
            pltpu.SemaphoreType.DMA,
        ],
    )


@jax.jit
def kernel(rois, probs, deltas, masks, window):
    probs_t = probs.transpose(2, 0, 1)
    deltas_t = deltas.transpose(0, 2, 3, 1)
    rois_t = rois.transpose(0, 2, 1)

    det_out, top_out = _tc_call(probs_t, deltas_t, rois_t, window)

    dets = det_out[:, :MAX_OUT, :6]
    nvalid = det_out[:, MAX_OUT, 0].astype(jnp.int32)
    tsel = top_out[:, 0, :]
    scale = (
        jnp.arange(MAX_OUT, dtype=jnp.int32)[None, :] < nvalid[:, None]
    ).astype(jnp.float32)

    idx_chunks = jnp.stack(
        [tsel[b, o:o + 16] for b in range(B) for o in _OFFS])
    scl_chunks = jnp.stack(
        [scale[b, o:o + 16] for b in range(B) for o in _OFFS])

    src = masks.transpose(2, 3, 4, 0, 1).reshape(MROW, B, N)
    out3 = _sc_gather_fn()(src, idx_chunks, scl_chunks)
    mk = out3.reshape(MH, MW, C, B, MAX_OUT).transpose(3, 4, 0, 1, 2)
    return dets, mk


# speedup vs baseline: 235.1089x; 1.2207x over previous
import functools

import jax
import jax.numpy as jnp
from jax import lax
from jax.experimental import pallas as pl
from jax.experimental.pallas import tpu as pltpu
from jax.experimental.pallas import tpu_sc as plsc

B = 2
N = 1000
C = 81
MH = 14
MW = 14
MROW = MH * MW * C
MAX_OUT = 100
NMS_THR = 0.3
MIN_CONF = 0.5
STD = (0.1, 0.1, 0.2, 0.2)


def _tc_body(win_ref, probs_ref, deltas_ref, rois_ref, det_ref, top_ref):
    wy1 = win_ref[0]
    wx1 = win_ref[1]
    wy2 = win_ref[2]
    wx2 = win_ref[3]

    best = probs_ref[0]
    cid = jnp.zeros((B, N), jnp.int32)
    d0 = deltas_ref[:, 0, 0, :]
    d1 = deltas_ref[:, 0, 1, :]
    d2 = deltas_ref[:, 0, 2, :]
    d3 = deltas_ref[:, 0, 3, :]
    for c in range(1, C):
        pc = probs_ref[c]
        upd = pc > best
        best = jnp.where(upd, pc, best)
        cid = jnp.where(upd, c, cid)
        d0 = jnp.where(upd, deltas_ref[:, c, 0, :], d0)
        d1 = jnp.where(upd, deltas_ref[:, c, 1, :], d1)
        d2 = jnp.where(upd, deltas_ref[:, c, 2, :], d2)
        d3 = jnp.where(upd, deltas_ref[:, c, 3, :], d3)

    ry1 = rois_ref[:, 0, :]
    rx1 = rois_ref[:, 1, :]
    ry2 = rois_ref[:, 2, :]
    rx2 = rois_ref[:, 3, :]
    nz = (jnp.abs(ry1) + jnp.abs(rx1) + jnp.abs(ry2) + jnp.abs(rx2)) != 0.0

    height = ry2 - ry1
    width = rx2 - rx1
    cy = ry1 + 0.5 * height + (d0 * STD[0]) * height
    cx = rx1 + 0.5 * width + (d1 * STD[1]) * width
    height = height * jnp.exp(d2 * STD[2])
    width = width * jnp.exp(d3 * STD[3])
    y1 = cy - 0.5 * height
    x1 = cx - 0.5 * width
    y2 = y1 + height
    x2 = x1 + width
    y1 = jnp.clip(y1, wy1, wy2)
    x1 = jnp.clip(x1, wx1, wx2)
    y2 = jnp.clip(y2, wy1, wy2)
    x2 = jnp.clip(x2, wx1, wx2)
    area = (y2 - y1) * (x2 - x1)

    valid = nz & (cid > 0) & (best >= MIN_CONF)
    key0 = jnp.where(valid, best, -1.0)

    lin = lax.broadcasted_iota(jnp.int32, (B, N), 1)
    lane = lax.broadcasted_iota(jnp.int32, (B, 1, 128), 2)

    def step(t, carry):
        key, top, cnt = carry
        m = jnp.max(key, axis=1, keepdims=True)
        alive = m >= 0.0
        idx = jnp.min(jnp.where(key == m, lin, N), axis=1, keepdims=True)
        sel = lin == idx
        gy1 = jnp.sum(jnp.where(sel, y1, 0.0), axis=1, keepdims=True)
        gx1 = jnp.sum(jnp.where(sel, x1, 0.0), axis=1, keepdims=True)
        gy2 = jnp.sum(jnp.where(sel, y2, 0.0), axis=1, keepdims=True)
        gx2 = jnp.sum(jnp.where(sel, x2, 0.0), axis=1, keepdims=True)
        ga = jnp.sum(jnp.where(sel, area, 0.0), axis=1, keepdims=True)
        gs = jnp.sum(jnp.where(sel, best, 0.0), axis=1, keepdims=True)
        gc = jnp.sum(jnp.where(sel, cid, 0), axis=1, keepdims=True)
        yy1 = jnp.maximum(gy1, y1)
        xx1 = jnp.maximum(gx1, x1)
        yy2 = jnp.minimum(gy2, y2)
        xx2 = jnp.minimum(gx2, x2)
        inter = jnp.maximum(0.0, yy2 - yy1) * jnp.maximum(0.0, xx2 - xx1)
        iou = inter / (ga + area - inter + 1e-12)
        supp = (cid == gc) & (iou > NMS_THR)
        key = jnp.where(alive, jnp.where(supp | sel, -1.0, key), key)
        top = jnp.where(alive & (lin == t), idx, top)
        cnt = cnt + jnp.where(alive, 1, 0)
        row = jnp.where(
            lane == 0, gy1[:, :, None],
            jnp.where(lane == 1, gx1[:, :, None],
                      jnp.where(lane == 2, gy2[:, :, None],
                                jnp.where(lane == 3, gx2[:, :, None],
                                          jnp.where(lane == 4,
                                                    gc.astype(jnp.float32)[:, :, None],
                                                    jnp.where(lane == 5,
                                                              gs[:, :, None], 0.0))))))
        row = jnp.where(alive[:, :, None], row, 0.0)
        det_ref[:, pl.ds(t, 1), :] = row
        return key, top, cnt

    top0 = jnp.zeros((B, N), jnp.int32)
    cnt0 = jnp.zeros((B, 1), jnp.int32)
    _, top, cnt = lax.fori_loop(0, MAX_OUT, step, (key0, top0, cnt0))
    det_ref[:, pl.ds(MAX_OUT, 1), :] = (
        cnt.astype(jnp.float32)[:, :, None] + jnp.zeros((B, 1, 128), jnp.float32)
    )
    top_ref[:, pl.ds(0, 1), :] = top[:, None, :]


def _tc_call(probs_t, deltas_t, rois_t, window):
    return pl.pallas_call(
        _tc_body,
        in_specs=[
            pl.BlockSpec(memory_space=pltpu.SMEM),
            pl.BlockSpec((C, B, N), lambda: (0, 0, 0)),
            pl.BlockSpec((B, C, 4, N), lambda: (0, 0, 0, 0)),
            pl.BlockSpec((B, 4, N), lambda: (0, 0, 0)),
        ],
        out_specs=[
            pl.BlockSpec((B, MAX_OUT + 1, 128), lambda: (0, 0, 0)),
            pl.BlockSpec((B, 1, N), lambda: (0, 0, 0)),
        ],
        out_shape=[
            jax.ShapeDtypeStruct((B, MAX_OUT + 1, 128), jnp.float32),
            jax.ShapeDtypeStruct((B, 1, N), jnp.int32),
        ],
    )(window, probs_t, deltas_t, rois_t)


_OFFS = (0, 16, 32, 48, 64, 80, 96)
_NCH = len(_OFFS)
_OW = 112
_BLK = 8
_NBLK = MROW // _BLK
_BPT = _NBLK // 32


def _sc_body(src, idx_hbm, scl_hbm, out, idxv, sclv,
             i00, i01, i10, i11, o00, o01, o10, o11,
             si00, si01, si10, si11, so00, so01, so10, so11):
    wid = lax.axis_index("s") * 2 + lax.axis_index("c")
    pltpu.sync_copy(idx_hbm, idxv)
    pltpu.sync_copy(scl_hbm, sclv)
    zero = jnp.zeros((16,), jnp.int32)
    ibs = ((i00, i01), (i10, i11))
    obs = ((o00, o01), (o10, o11))
    sis = ((si00, si01), (si10, si11))
    sos = ((so00, so01), (so10, so11))

    def process(ph, nrows):
        for b in range(B):
            ib = ibs[ph][b]
            ob = obs[ph][b]
            for r in range(nrows):
                rv = zero + r
                for j in range(_NCH):
                    g = plsc.load_gather(ib, [rv, idxv[b * _NCH + j]])
                    ob[r, pl.ds(_OFFS[j], 16)] = g * sclv[b * _NCH + j]

    base0 = wid * _BPT * _BLK
    for b in range(B):
        pltpu.make_async_copy(
            src.at[pl.ds(base0, _BLK), b], ibs[0][b], sis[0][b]).start()
        pltpu.make_async_copy(
            src.at[pl.ds(base0 + _BLK, _BLK), b], ibs[1][b], sis[1][b]).start()

    def outer(i, _):
        for ph in (0, 1):
            j = 2 * i + ph
            rbase = base0 + j * _BLK
            for b in range(B):
                pltpu.make_async_copy(
                    src.at[pl.ds(rbase, _BLK), b], ibs[ph][b], sis[ph][b]).wait()

            @pl.when(j >= 2)
            def _():
                for b in range(B):
                    pltpu.make_async_copy(
                        obs[ph][b], out.at[pl.ds(rbase - 2 * _BLK, _BLK), b],
                        sos[ph][b]).wait()

            process(ph, _BLK)
            for b in range(B):
                pltpu.make_async_copy(
                    obs[ph][b], out.at[pl.ds(rbase, _BLK), b], sos[ph][b]).start()

            @pl.when(j + 2 < _BPT)
            def _():
                for b in range(B):
                    pltpu.make_async_copy(
                        src.at[pl.ds(rbase + 2 * _BLK, _BLK), b],
                        ibs[ph][b], sis[ph][b]).start()

        return 0

    lax.fori_loop(0, _BPT // 2, outer, 0)
    last = base0 + (_BPT - 2) * _BLK
    for ph in (0, 1):
        for b in range(B):
            pltpu.make_async_copy(
                obs[ph][b], out.at[pl.ds(last + ph * _BLK, _BLK), b],
                sos[ph][b]).wait()

    @pl.when(wid == 0)
    def _():
        tbase = 32 * _BPT * _BLK
        for b in range(B):
            pltpu.sync_copy(src.at[pl.ds(tbase, 4), b], ibs[0][b].at[pl.ds(0, 4)])
        process(0, 4)
        for b in range(B):
            pltpu.sync_copy(obs[0][b].at[pl.ds(0, 4)], out.at[pl.ds(tbase, 4), b])


@functools.lru_cache(maxsize=1)
def _sc_gather_fn():
    return pl.kernel(
        _sc_body,
        out_type=jax.ShapeDtypeStruct((MROW, B, _OW), jnp.float32),
        mesh=plsc.VectorSubcoreMesh(core_axis_name="c", subcore_axis_name="s"),
        compiler_params=pltpu.CompilerParams(needs_layout_passes=False),
        scratch_types=[
            pltpu.VMEM((B * _NCH, 16), jnp.int32),
            pltpu.VMEM((B * _NCH, 16), jnp.float32),
            pltpu.VMEM((_BLK, N), jnp.float32),
            pltpu.VMEM((_BLK, N), jnp.float32),
            pltpu.VMEM((_BLK, N), jnp.float32),
            pltpu.VMEM((_BLK, N), jnp.float32),
            pltpu.VMEM((_BLK, _OW), jnp.float32),
            pltpu.VMEM((_BLK, _OW), jnp.float32),
            pltpu.VMEM((_BLK, _OW), jnp.float32),
            pltpu.VMEM((_BLK, _OW), jnp.float32),
            pltpu.SemaphoreType.DMA,
            pltpu.SemaphoreType.DMA,
            pltpu.SemaphoreType.DMA,
            pltpu.SemaphoreType.DMA,
            pltpu.SemaphoreType.DMA,
            pltpu.SemaphoreType.DMA,
            pltpu.SemaphoreType.DMA,
            pltpu.SemaphoreType.DMA,
        ],
    )


@jax.jit
def kernel(rois, probs, deltas, masks, window):
    probs_t = probs.transpose(2, 0, 1)
    deltas_t = deltas.transpose(0, 2, 3, 1)
    rois_t = rois.transpose(0, 2, 1)

    det_out, top_out = _tc_call(probs_t, deltas_t, rois_t, window)

    dets = det_out[:, :MAX_OUT, :6]
    nvalid = det_out[:, MAX_OUT, 0].astype(jnp.int32)
    tsel = top_out[:, 0, :_OW]
    scale = (
        jnp.arange(_OW, dtype=jnp.int32)[None, :] < nvalid[:, None]
    ).astype(jnp.float32)

    idx_chunks = jnp.stack(
        [tsel[b, o:o + 16] for b in range(B) for o in _OFFS])
    scl_chunks = jnp.stack(
        [scale[b, o:o + 16] for b in range(B) for o in _OFFS])

    src = masks.transpose(2, 3, 4, 0, 1).reshape(MROW, B, N)
    out3 = _sc_gather_fn()(src, idx_chunks, scl_chunks)
    mk = (
        out3[:, :, :MAX_OUT]
        .reshape(MH, MW, C, B, MAX_OUT)
        .transpose(3, 4, 0, 1, 2)
    )
    return dets, mk
